# baseline (speedup 1.0000x reference)
import jax
import jax.numpy as jnp
from jax import lax
from jax.experimental import pallas as pl
from jax.experimental.pallas import tpu as pltpu
from jax.experimental.pallas import tpu_sc as plsc

HEAD_DIM = 64
Q_LEN = 32
K_LEN = 8192
WIN = 544
TAIL = K_LEN - WIN
PAD_ROWS = 1056
TAB_BASE = 480
TAB_ROWS = PAD_ROWS - TAB_BASE


def _sc_head_body(padded_hbm, head_hbm, tab, sem):
    c = lax.axis_index("c")
    s = lax.axis_index("s")
    q = s * 2 + c
    pltpu.async_copy(padded_hbm.at[pl.ds(TAB_BASE, TAB_ROWS)], tab, sem).wait()
    pltpu.async_copy(tab.at[pl.ds(512 - TAB_BASE - q, WIN)],
                     head_hbm.at[q], sem).wait()


QB = 8


def _tc_assemble_body(head_ref, col_ref, out_ref):
    for i in range(QB):
        out_ref[i, :, :] = jnp.broadcast_to(col_ref[...], (HEAD_DIM, K_LEN))
        out_ref[i, :, 0:WIN] = jnp.swapaxes(head_ref[i], 0, 1)


def kernel(query_len, key_len, pos_embed):
    del query_len, key_len
    pad = jnp.broadcast_to(pos_embed[-1], (PAD_ROWS - 1025, HEAD_DIM))
    padded = jnp.concatenate([pos_embed, pad], axis=0)

    mesh = plsc.VectorSubcoreMesh(core_axis_name="c", subcore_axis_name="s")
    head_fn = pl.kernel(
        _sc_head_body,
        out_type=jax.ShapeDtypeStruct((Q_LEN, WIN, HEAD_DIM), jnp.float32),
        mesh=mesh,
        scratch_types=[
            pltpu.VMEM((TAB_ROWS, HEAD_DIM), jnp.float32),
            pltpu.SemaphoreType.DMA,
        ],
    )
    head = head_fn(padded)

    col = pos_embed[-1][:, None]
    out_t = pl.pallas_call(
        _tc_assemble_body,
        grid=(Q_LEN // QB,),
        out_shape=jax.ShapeDtypeStruct((Q_LEN, HEAD_DIM, K_LEN), jnp.float32),
        in_specs=[
            pl.BlockSpec((QB, WIN, HEAD_DIM), lambda q: (q, 0, 0)),
            pl.BlockSpec((HEAD_DIM, 1), lambda q: (0, 0)),
        ],
        out_specs=pl.BlockSpec((QB, HEAD_DIM, K_LEN), lambda q: (q, 0, 0)),
    )(head, col)
    return jnp.transpose(out_t, (0, 2, 1))


# speedup vs baseline: 1.0125x; 1.0125x over previous
import jax
import jax.numpy as jnp
from jax import lax
from jax.experimental import pallas as pl
from jax.experimental.pallas import tpu as pltpu
from jax.experimental.pallas import tpu_sc as plsc

HEAD_DIM = 64
Q_LEN = 32
K_LEN = 8192
WIN = 544
TAIL = K_LEN - WIN
PAD_ROWS = 1056
TAB_BASE = 480
TAB_ROWS = PAD_ROWS - TAB_BASE


def _sc_head_body(padded_hbm, head_hbm, tab, sem):
    c = lax.axis_index("c")
    s = lax.axis_index("s")
    q = s * 2 + c
    pltpu.async_copy(padded_hbm.at[pl.ds(TAB_BASE, TAB_ROWS)], tab, sem).wait()
    pltpu.async_copy(tab.at[pl.ds(512 - TAB_BASE - q, WIN)],
                     head_hbm.at[q], sem).wait()


QB = 4


def _tc_assemble_body(head_ref, col_ref, out_ref):
    for i in range(QB):
        out_ref[i, :, :] = jnp.broadcast_to(col_ref[...], (HEAD_DIM, K_LEN))
        out_ref[i, :, 0:WIN] = jnp.swapaxes(head_ref[i], 0, 1)


def kernel(query_len, key_len, pos_embed):
    del query_len, key_len
    pad = jnp.broadcast_to(pos_embed[-1], (PAD_ROWS - 1025, HEAD_DIM))
    padded = jnp.concatenate([pos_embed, pad], axis=0)

    mesh = plsc.VectorSubcoreMesh(core_axis_name="c", subcore_axis_name="s")
    head_fn = pl.kernel(
        _sc_head_body,
        out_type=jax.ShapeDtypeStruct((Q_LEN, WIN, HEAD_DIM), jnp.float32),
        mesh=mesh,
        scratch_types=[
            pltpu.VMEM((TAB_ROWS, HEAD_DIM), jnp.float32),
            pltpu.SemaphoreType.DMA,
        ],
    )
    head = head_fn(padded)

    col = pos_embed[-1][:, None]
    out_t = pl.pallas_call(
        _tc_assemble_body,
        grid=(Q_LEN // QB,),
        out_shape=jax.ShapeDtypeStruct((Q_LEN, HEAD_DIM, K_LEN), jnp.float32),
        in_specs=[
            pl.BlockSpec((QB, WIN, HEAD_DIM), lambda q: (q, 0, 0)),
            pl.BlockSpec((HEAD_DIM, 1), lambda q: (0, 0)),
        ],
        out_specs=pl.BlockSpec((QB, HEAD_DIM, K_LEN), lambda q: (q, 0, 0)),
    )(head, col)
    return jnp.transpose(out_t, (0, 2, 1))


# speedup vs baseline: 1.1423x; 1.1282x over previous
import jax
import jax.numpy as jnp
from jax import lax
from jax.experimental import pallas as pl
from jax.experimental.pallas import tpu as pltpu
from jax.experimental.pallas import tpu_sc as plsc

HEAD_DIM = 64
Q_LEN = 32
K_LEN = 8192
WIN = 544
TAIL = K_LEN - WIN
PAD_ROWS = 1056
TAB_BASE = 480
TAB_ROWS = PAD_ROWS - TAB_BASE


def _sc_head_body(padded_hbm, head_hbm, tab_shared):
    c = lax.axis_index("c")
    s = lax.axis_index("s")
    q = s * 2 + c

    @pl.when(s == 0)
    def _():
        pltpu.sync_copy(padded_hbm.at[pl.ds(TAB_BASE, TAB_ROWS)], tab_shared)

    plsc.subcore_barrier()
    pltpu.sync_copy(tab_shared.at[pl.ds(512 - TAB_BASE - q, WIN)],
                    head_hbm.at[q])


QB = 4


def _tc_assemble_body(head_ref, col_ref, out_ref):
    for i in range(QB):
        out_ref[i, :, :] = jnp.broadcast_to(col_ref[...], (HEAD_DIM, K_LEN))
        out_ref[i, :, 0:WIN] = jnp.swapaxes(head_ref[i], 0, 1)


def kernel(query_len, key_len, pos_embed):
    del query_len, key_len
    pad = jnp.broadcast_to(pos_embed[-1], (PAD_ROWS - 1025, HEAD_DIM))
    padded = jnp.concatenate([pos_embed, pad], axis=0)

    mesh = plsc.VectorSubcoreMesh(core_axis_name="c", subcore_axis_name="s")
    head_fn = pl.kernel(
        _sc_head_body,
        out_type=jax.ShapeDtypeStruct((Q_LEN, WIN, HEAD_DIM), jnp.float32),
        mesh=mesh,
        scratch_types=[
            pltpu.VMEM_SHARED((TAB_ROWS, HEAD_DIM), jnp.float32),
        ],
    )
    head = head_fn(padded)

    col = pos_embed[-1][:, None]
    out_t = pl.pallas_call(
        _tc_assemble_body,
        grid=(Q_LEN // QB,),
        out_shape=jax.ShapeDtypeStruct((Q_LEN, HEAD_DIM, K_LEN), jnp.float32),
        in_specs=[
            pl.BlockSpec((QB, WIN, HEAD_DIM), lambda q: (q, 0, 0)),
            pl.BlockSpec((HEAD_DIM, 1), lambda q: (0, 0)),
        ],
        out_specs=pl.BlockSpec((QB, HEAD_DIM, K_LEN), lambda q: (q, 0, 0)),
    )(head, col)
    return jnp.transpose(out_t, (0, 2, 1))


# speedup vs baseline: 1.1570x; 1.0129x over previous
import jax
import jax.numpy as jnp
from jax import lax
from jax.experimental import pallas as pl
from jax.experimental.pallas import tpu as pltpu
from jax.experimental.pallas import tpu_sc as plsc

HEAD_DIM = 64
Q_LEN = 32
K_LEN = 8192
WIN = 544
TAIL = K_LEN - WIN
PAD_ROWS = 1056
TAB_BASE = 480
TAB_ROWS = PAD_ROWS - TAB_BASE


def _sc_head_body(padded_hbm, head_hbm, col_hbm, tab_shared):
    c = lax.axis_index("c")
    s = lax.axis_index("s")
    q = s * 2 + c

    @pl.when(s == 0)
    def _():
        pltpu.sync_copy(padded_hbm.at[pl.ds(TAB_BASE, TAB_ROWS)], tab_shared)

    plsc.subcore_barrier()
    pltpu.sync_copy(tab_shared.at[pl.ds(512 - TAB_BASE - q, WIN)],
                    head_hbm.at[q])

    @pl.when(q == 0)
    def _():
        pltpu.sync_copy(tab_shared.at[pl.ds(544, 8)], col_hbm)


QB = 4


def _tc_assemble_body(head_ref, col_ref, out_ref):
    colv = jnp.swapaxes(col_ref[0:1, :], 0, 1)
    for i in range(QB):
        out_ref[i, :, :] = jnp.broadcast_to(colv, (HEAD_DIM, K_LEN))
        out_ref[i, :, 0:WIN] = jnp.swapaxes(head_ref[i], 0, 1)


def kernel(query_len, key_len, pos_embed):
    del query_len, key_len
    pad = jnp.broadcast_to(pos_embed[-1], (PAD_ROWS - 1025, HEAD_DIM))
    padded = jnp.concatenate([pos_embed, pad], axis=0)

    mesh = plsc.VectorSubcoreMesh(core_axis_name="c", subcore_axis_name="s")
    head_fn = pl.kernel(
        _sc_head_body,
        out_type=(
            jax.ShapeDtypeStruct((Q_LEN, WIN, HEAD_DIM), jnp.float32),
            jax.ShapeDtypeStruct((8, HEAD_DIM), jnp.float32),
        ),
        mesh=mesh,
        scratch_types=[
            pltpu.VMEM_SHARED((TAB_ROWS, HEAD_DIM), jnp.float32),
        ],
    )
    head, col = head_fn(padded)

    out_t = pl.pallas_call(
        _tc_assemble_body,
        grid=(Q_LEN // QB,),
        out_shape=jax.ShapeDtypeStruct((Q_LEN, HEAD_DIM, K_LEN), jnp.float32),
        in_specs=[
            pl.BlockSpec((QB, WIN, HEAD_DIM), lambda q: (q, 0, 0)),
            pl.BlockSpec((8, HEAD_DIM), lambda q: (0, 0)),
        ],
        out_specs=pl.BlockSpec((QB, HEAD_DIM, K_LEN), lambda q: (q, 0, 0)),
    )(head, col)
    return jnp.transpose(out_t, (0, 2, 1))
